# Initial kernel scaffold; baseline (speedup 1.0000x reference)
import jax
import jax.numpy as jnp
from jax.experimental import pallas as pl


def kernel(wordid, mask, h0, c0, emb_table, W_iou, U_iou, b_iou, W_f, b_f):
    raise NotImplementedError("write your pallas kernel here")


# speedup vs baseline: 2.5048x; 2.5048x over previous
import functools

import jax
import jax.numpy as jnp
import numpy as np
from jax import lax
from jax.experimental import pallas as pl
from jax.experimental.pallas import tpu as pltpu

B = 64
D = 10
NPT = 2 ** D - 1
H = 128
E = 128
L = 5
NLEAF = 2 ** (D - 1)
G = 8


def _tree_body(x_ref, mask_ref, Wiou_ref, Uiou_ref, biou_ref, Wf_ref, bf_ref,
               h_ref, rooth_ref, rootc_ref, c_ref):
    Wiou = Wiou_ref[...]
    Uiou = Uiou_ref[...]
    biou = biou_ref[...]
    Wf = Wf_ref[...]
    bf = bf_ref[...]

    x = x_ref[...].reshape(G * NLEAF, E) * mask_ref[...]
    iou = jnp.dot(x, Wiou, preferred_element_type=jnp.float32) + biou
    ig = iou[:, :H]
    og = iou[:, H:2 * H]
    ug = iou[:, 2 * H:]
    c_new = jax.nn.sigmoid(ig) * jnp.tanh(ug)
    h_new = jax.nn.sigmoid(og) * jnp.tanh(c_new)
    h_ref[:, NLEAF - 1:NPT, :] = h_new.reshape(G, NLEAF, H)
    c_ref[:, NLEAF - 1:NPT, :] = c_new.reshape(G, NLEAF, H)

    for l in range(D - 2, -1, -1):
        r = 2 ** l
        ps = r - 1
        cs = 2 * r - 1
        hc = h_ref[:, cs:cs + 2 * r, :].reshape(G * 2 * r, H)
        cc = c_ref[:, cs:cs + 2 * r, :].reshape(G * 2 * r, H)
        f = jax.nn.sigmoid(
            lax.dot_general(hc, Wf, (((1,), (1,)), ((), ())),
                            preferred_element_type=jnp.float32) + bf)
        c_in = (f * cc).reshape(G * r, 2, H).sum(axis=1)
        hs = hc.reshape(G * r, 2, H).sum(axis=1)
        iou = jnp.dot(hs, Uiou, preferred_element_type=jnp.float32) + biou
        ig = iou[:, :H]
        og = iou[:, H:2 * H]
        ug = iou[:, 2 * H:]
        c_new = jax.nn.sigmoid(ig) * jnp.tanh(ug) + c_in
        h_new = jax.nn.sigmoid(og) * jnp.tanh(c_new)
        h_ref[:, ps:ps + r, :] = h_new.reshape(G, r, H)
        c_ref[:, ps:ps + r, :] = c_new.reshape(G, r, H)

    rooth_ref[...] = h_new
    rootc_ref[...] = c_new


def _tree_sweep(x_leaf, mask_leaf_f, W_iou, U_iou, b_iou, W_f, b_f2,
                interpret=False):
    grid = (B // G,)
    out_shapes = (
        jax.ShapeDtypeStruct((B, NPT, H), jnp.float32),
        jax.ShapeDtypeStruct((B, H), jnp.float32),
        jax.ShapeDtypeStruct((B, H), jnp.float32),
    )
    return pl.pallas_call(
        _tree_body,
        grid=grid,
        in_specs=[
            pl.BlockSpec((G, NLEAF, E), lambda i: (i, 0, 0)),
            pl.BlockSpec((G * NLEAF, 1), lambda i: (i, 0)),
            pl.BlockSpec((E, 3 * H), lambda i: (0, 0)),
            pl.BlockSpec((H, 3 * H), lambda i: (0, 0)),
            pl.BlockSpec((1, 3 * H), lambda i: (0, 0)),
            pl.BlockSpec((H, H), lambda i: (0, 0)),
            pl.BlockSpec((1, H), lambda i: (0, 0)),
        ],
        out_specs=(
            pl.BlockSpec((G, NPT, H), lambda i: (i, 0, 0)),
            pl.BlockSpec((G, H), lambda i: (i, 0)),
            pl.BlockSpec((G, H), lambda i: (i, 0)),
        ),
        out_shape=out_shapes,
        scratch_shapes=[pltpu.VMEM((G, NPT, H), jnp.float32)],
        interpret=interpret,
    )(x_leaf, mask_leaf_f, W_iou, U_iou, b_iou, W_f, b_f2)


def _impl(wordid, mask, h0, c0, emb_table, W_iou, U_iou, b_iou, W_f, b_f,
          interpret=False):
    ids = (wordid * mask[:, None]).reshape(B, NPT, L)[:, NLEAF - 1:, :]
    mask_leaf = mask.reshape(B, NPT)[:, NLEAF - 1:].astype(
        jnp.float32).reshape(B * NLEAF, 1)
    x_leaf = jnp.sum(jnp.take(emb_table, ids.reshape(-1, L), axis=0), axis=1)
    x_leaf = x_leaf.reshape(B, NLEAF, E)
    tree_output, root_h, root_c = _tree_sweep(
        x_leaf, mask_leaf, W_iou, U_iou, b_iou, W_f,
        b_f.reshape(1, H), interpret=interpret)
    return tree_output, root_h, root_c


def kernel(wordid, mask, h0, c0, emb_table, W_iou, U_iou, b_iou, W_f, b_f):
    return _impl(wordid, mask, h0, c0, emb_table, W_iou, U_iou, b_iou,
                 W_f, b_f)
